# baseline (speedup 1.0000x reference)
import functools

import jax
import jax.numpy as jnp
from jax import lax
from jax.experimental import pallas as pl


def _pool_body(x_ref, o_ref, *, inv_n):
    o_ref[...] = jnp.sum(x_ref[...], axis=1, keepdims=True) * inv_n


def _sim_agg_body(pool_ref, agg_ref):
    p = pool_ref[...]
    b = p.shape[0]
    s2 = jnp.sum(p * p, axis=1, keepdims=True)
    norm = jnp.sqrt(s2)
    xn = p / jnp.maximum(norm, 1e-12)
    s = jnp.dot(xn, xn.T, preferred_element_type=jnp.float32)
    rows = lax.broadcasted_iota(jnp.int32, (b, b), 0)
    cols = lax.broadcasted_iota(jnp.int32, (b, b), 1)
    s = jnp.where(rows == cols, 0.0, s)
    maxv = jnp.max(s, axis=1, keepdims=True)
    cand = jnp.where(s == maxv, cols, b)
    amin = jnp.min(cand, axis=1, keepdims=True)
    attn = jnp.where(cols == amin, maxv, 0.0)
    agg_ref[...] = jnp.dot(attn, p, preferred_element_type=jnp.float32)


def _add_body(x_ref, agg_ref, o_ref):
    o_ref[...] = x_ref[...] + agg_ref[...]


def kernel(x):
    b, n, c = x.shape
    pool = pl.pallas_call(
        functools.partial(_pool_body, inv_n=1.0 / n),
        grid=(b,),
        in_specs=[pl.BlockSpec((1, n, c), lambda i: (i, 0, 0))],
        out_specs=pl.BlockSpec((1, 1, c), lambda i: (i, 0, 0)),
        out_shape=jax.ShapeDtypeStruct((b, 1, c), jnp.float32),
    )(x)

    agg = pl.pallas_call(
        _sim_agg_body,
        out_shape=jax.ShapeDtypeStruct((b, c), jnp.float32),
    )(pool.reshape(b, c))

    out = pl.pallas_call(
        _add_body,
        grid=(b,),
        in_specs=[
            pl.BlockSpec((1, n, c), lambda i: (i, 0, 0)),
            pl.BlockSpec((1, 1, c), lambda i: (i, 0, 0)),
        ],
        out_specs=pl.BlockSpec((1, n, c), lambda i: (i, 0, 0)),
        out_shape=jax.ShapeDtypeStruct((b, n, c), jnp.float32),
    )(x, agg.reshape(b, 1, c))
    return out


# speedup vs baseline: 1.0108x; 1.0108x over previous
import functools

import jax
import jax.numpy as jnp
from jax import lax
from jax.experimental import pallas as pl
from jax.experimental.pallas import tpu as pltpu


def _pool_sim_body(x_ref, agg_ref, acc_ref, *, b, inv_n):
    i = pl.program_id(0)
    row = jnp.sum(x_ref[...], axis=1) * inv_n
    acc_ref[pl.ds(i, 1), :] = row

    @pl.when(i == b - 1)
    def _():
        p = acc_ref[...]
        s2 = jnp.sum(p * p, axis=1, keepdims=True)
        norm = jnp.sqrt(s2)
        xn = p / jnp.maximum(norm, 1e-12)
        s = jnp.dot(xn, xn.T, preferred_element_type=jnp.float32)
        rows = lax.broadcasted_iota(jnp.int32, (b, b), 0)
        cols = lax.broadcasted_iota(jnp.int32, (b, b), 1)
        s = jnp.where(rows == cols, 0.0, s)
        maxv = jnp.max(s, axis=1, keepdims=True)
        cand = jnp.where(s == maxv, cols, b)
        amin = jnp.min(cand, axis=1, keepdims=True)
        attn = jnp.where(cols == amin, maxv, 0.0)
        agg_ref[...] = jnp.dot(attn, p, preferred_element_type=jnp.float32)


def _add_body(x_ref, agg_ref, o_ref):
    o_ref[...] = x_ref[...] + agg_ref[...]


def kernel(x):
    b, n, c = x.shape
    agg = pl.pallas_call(
        functools.partial(_pool_sim_body, b=b, inv_n=1.0 / n),
        grid=(b,),
        in_specs=[pl.BlockSpec((1, n, c), lambda i: (i, 0, 0))],
        out_specs=pl.BlockSpec((b, c), lambda i: (0, 0)),
        out_shape=jax.ShapeDtypeStruct((b, c), jnp.float32),
        scratch_shapes=[pltpu.VMEM((b, c), jnp.float32)],
    )(x)

    out = pl.pallas_call(
        _add_body,
        grid=(b,),
        in_specs=[
            pl.BlockSpec((1, n, c), lambda i: (i, 0, 0)),
            pl.BlockSpec((1, 1, c), lambda i: (i, 0, 0)),
        ],
        out_specs=pl.BlockSpec((1, n, c), lambda i: (i, 0, 0)),
        out_shape=jax.ShapeDtypeStruct((b, n, c), jnp.float32),
    )(x, agg.reshape(b, 1, c))
    return out
